# Initial kernel scaffold; baseline (speedup 1.0000x reference)
import jax
import jax.numpy as jnp
from jax.experimental import pallas as pl


def kernel(activations, cache, n_valid, index):
    raise NotImplementedError("write your pallas kernel here")


# speedup vs baseline: 1.7426x; 1.7426x over previous
import functools

import jax
import jax.numpy as jnp
from jax.experimental import pallas as pl
from jax.experimental.pallas import tpu as pltpu

_R2 = 4000


def _merge_body(idx_ref, cache_ref, act_hbm, out_ref, act_vmem, sem, *, M, B, R2, P2):
    i = pl.program_id(0)
    index = idx_ref[0]
    delta = index % 16
    a1 = 2 * (i * R2) - index
    a1 = jnp.where(a1 < 0, a1 + M, a1)
    overlap = (a1 < B) | (a1 > M - 2 * R2)

    out_ref[...] = cache_ref[...]

    @pl.when(overlap)
    def _():
        anum = jnp.where(a1 < B, a1 + delta, a1 + delta - M)
        base = pl.multiple_of(P2 + anum // 2, 8)
        cp = pltpu.make_async_copy(act_hbm.at[pl.ds(base, R2)], act_vmem, sem)
        cp.start()
        cp.wait()
        h = (jax.lax.broadcasted_iota(jnp.int32, (1, 128), 1) >= 64).astype(jnp.int32)
        off = a1 + 2 * jax.lax.broadcasted_iota(jnp.int32, (R2, 1), 0) + h
        off = jnp.where(off >= M, off - M, off)
        out_ref[...] = jnp.where(off < B, act_vmem[...], out_ref[...])


def kernel(activations, cache, n_valid, index):
    M, N = cache.shape
    B = activations.shape[0]
    R2 = _R2
    M2 = M // 2
    assert M2 % R2 == 0 and N == 64 and B % 2 == 0
    num_blocks = M2 // R2
    P2 = R2
    L2 = 2 * R2 + B // 2 + 16

    idx = jnp.asarray(index, jnp.int32) % M
    act16 = jax.lax.bitcast_convert_type(
        activations.astype(cache.dtype), jnp.int16
    )
    cache_u = jax.lax.bitcast_convert_type(cache, jnp.int16)
    act_pad = jax.lax.dynamic_update_slice(
        jnp.zeros((2 * L2, N), jnp.int16), act16, (2 * P2 + idx % 16, 0)
    ).reshape(L2, 128)

    grid_spec = pltpu.PrefetchScalarGridSpec(
        num_scalar_prefetch=1,
        grid=(num_blocks,),
        in_specs=[
            pl.BlockSpec((R2, 128), lambda i, s: (i, 0)),
            pl.BlockSpec(memory_space=pltpu.MemorySpace.HBM),
        ],
        out_specs=pl.BlockSpec((R2, 128), lambda i, s: (i, 0)),
        scratch_shapes=[pltpu.VMEM((R2, 128), jnp.int16), pltpu.SemaphoreType.DMA],
    )
    out128 = pl.pallas_call(
        functools.partial(_merge_body, M=M, B=B, R2=R2, P2=P2),
        grid_spec=grid_spec,
        out_shape=jax.ShapeDtypeStruct((M2, 128), jnp.int16),
    )(jnp.reshape(idx, (1,)), cache_u.reshape(M2, 128), act_pad)

    new_cache = jax.lax.bitcast_convert_type(out128.reshape(M, N), cache.dtype)
    new_n_valid = jnp.minimum(jnp.asarray(n_valid) + B, M)
    new_index = (jnp.asarray(index) + B) % M
    return (new_cache, new_n_valid, new_index)


# speedup vs baseline: 2.4317x; 1.3954x over previous
import functools
import jax
import jax.numpy as jnp
from jax.experimental import pallas as pl
from jax.experimental.pallas import tpu as pltpu

_R = 8000


def _merge_body(idx_ref, cache_ref, act_hbm, out_ref, act_vmem, sem, *, M, B, R, P):
    i = pl.program_id(0)
    index = idx_ref[0]
    delta = index % 16
    a1 = i * R - index
    a1 = jnp.where(a1 < 0, a1 + M, a1)
    overlap = (a1 < B) | (a1 > M - R)

    out_ref[...] = cache_ref[...]

    @pl.when(overlap)
    def _():
        base = jnp.where(a1 < B, a1 + delta, a1 + delta - M) + P
        base = pl.multiple_of(base, 8)
        cp = pltpu.make_async_copy(act_hbm.at[pl.ds(base, R)], act_vmem, sem)
        cp.start()
        cp.wait()
        off = a1 + jax.lax.broadcasted_iota(jnp.int32, (R, 1), 0)
        off = jnp.where(off >= M, off - M, off)
        out_ref[...] = jnp.where(off < B, act_vmem[...], out_ref[...])


def kernel(activations, cache, n_valid, index):
    M, N = cache.shape
    B = activations.shape[0]
    R = _R
    num_blocks = M // R
    P = R
    L = 2 * R + B + 16

    idx = jnp.asarray(index, jnp.int32) % M
    act16 = jax.lax.bitcast_convert_type(activations.astype(cache.dtype), jnp.int16)
    cache_u = jax.lax.bitcast_convert_type(cache, jnp.int16)
    act_pad = jax.lax.dynamic_update_slice(
        jnp.zeros((L, N), jnp.int16), act16, (P + idx % 16, 0)
    )

    grid_spec = pltpu.PrefetchScalarGridSpec(
        num_scalar_prefetch=1,
        grid=(num_blocks,),
        in_specs=[
            pl.BlockSpec((R, N), lambda i, s: (i, 0)),
            pl.BlockSpec(memory_space=pltpu.MemorySpace.HBM),
        ],
        out_specs=pl.BlockSpec((R, N), lambda i, s: (i, 0)),
        scratch_shapes=[pltpu.VMEM((R, N), jnp.int16), pltpu.SemaphoreType.DMA],
    )
    out_u = pl.pallas_call(
        functools.partial(_merge_body, M=M, B=B, R=R, P=P),
        grid_spec=grid_spec,
        out_shape=jax.ShapeDtypeStruct((M, N), jnp.int16),
    )(jnp.reshape(idx, (1,)), cache_u, act_pad)

    new_cache = jax.lax.bitcast_convert_type(out_u, cache.dtype)
    new_n_valid = jnp.minimum(jnp.asarray(n_valid) + B, M)
    new_index = (jnp.asarray(index) + B) % M
    return (new_cache, new_n_valid, new_index)
